# baseline (speedup 1.0000x reference)
import functools

import jax
import jax.numpy as jnp
from jax import lax
from jax.experimental import pallas as pl
from jax.experimental.pallas import tpu as pltpu
from jax.experimental.pallas import tpu_sc as plsc

_B = 16384
_D = 64
_NEG = 10
_MARGIN = 0.8
_NC = 2
_NS = 16
_L = 16
_NW = _NC * _NS
_BW = _B // _NW
_G = _L
_NG = _BW // _G
_GN = _G * _NEG
_HN = _GN // 2


def _rsqrt(x):
    xi = lax.bitcast_convert_type(x, jnp.int32)
    yi = jnp.int32(0x5F3759DF) - (xi >> 1)
    y = lax.bitcast_convert_type(yi, jnp.float32)
    for _ in range(3):
        y = y * (1.5 - 0.5 * x * y * y)
    return y


def _make_sc_kernel():
    mesh = plsc.VectorSubcoreMesh(core_axis_name="c", subcore_axis_name="s")

    @functools.partial(
        pl.kernel,
        mesh=mesh,
        out_type=jax.ShapeDtypeStruct((_NW, _L), jnp.float32),
        compiler_params=pltpu.CompilerParams(
            needs_layout_passes=False,
            use_tc_tiling_on_sc=False,
        ),
        scratch_types=[
            pltpu.VMEM((_BW,), jnp.int32),
            pltpu.VMEM((_BW,), jnp.int32),
            pltpu.VMEM((_BW * _NEG,), jnp.int32),
            pltpu.VMEM((2, _G, _D), jnp.float32),
            pltpu.VMEM((2, _G, _D), jnp.float32),
            pltpu.VMEM((2, _GN, _D), jnp.float32),
            pltpu.VMEM((_L,), jnp.float32),
            pltpu.SemaphoreType.DMA((2,)),
        ],
    )
    def sc_kernel(u_hbm, i_hbm, n_hbm, uemb_hbm, iemb_hbm, out_hbm,
                  uidx, iidx, nidx, ubuf, ibuf, nbuf, pvec, sems):
        wid = lax.axis_index("s") * _NC + lax.axis_index("c")
        base = pl.multiple_of(wid * _BW, _BW)
        nbase = pl.multiple_of(wid * (_BW * _NEG), _BW * _NEG)

        pltpu.sync_copy(u_hbm.at[pl.ds(base, _BW)], uidx)
        pltpu.sync_copy(i_hbm.at[pl.ds(base, _BW)], iidx)
        pltpu.sync_copy(n_hbm.at[pl.ds(nbase, _BW * _NEG)], nidx)

        def dma_triples(g, slot):
            go = pl.multiple_of(g * _G, _G)
            no = pl.multiple_of(g * _GN, _HN)
            return (
                (uemb_hbm.at[uidx.at[pl.ds(go, _G)]], ubuf.at[slot]),
                (iemb_hbm.at[iidx.at[pl.ds(go, _G)]], ibuf.at[slot]),
                (iemb_hbm.at[nidx.at[pl.ds(no, _HN)]],
                 nbuf.at[slot, pl.ds(0, _HN)]),
                (iemb_hbm.at[nidx.at[pl.ds(no + _HN, _HN)]],
                 nbuf.at[slot, pl.ds(_HN, _HN)]),
            )

        def start_group(g, slot):
            for src, dst in dma_triples(g, slot):
                pltpu.async_copy(src, dst, sems.at[slot])

        def wait_group(g, slot):
            for src, dst in dma_triples(g, slot):
                pltpu.make_async_copy(src, dst, sems.at[slot]).wait()

        riota = lax.iota(jnp.int32, _L)
        nrow_idx = [riota * _NEG + n for n in range(_NEG)]
        zv = jnp.zeros((_L,), jnp.float32)

        def compute_group(slot, acc_pos, acc_neg):
            ub, ib, nb = ubuf.at[slot], ibuf.at[slot], nbuf.at[slot]

            def dbody(d, carry):
                ss_u, ss_i, d_ui, ss_n, d_un = carry
                cd = jnp.broadcast_to(d, (_L,)).astype(jnp.int32)
                uc = plsc.load_gather(ub, [riota, cd])
                ic = plsc.load_gather(ib, [riota, cd])
                ss_u = ss_u + uc * uc
                ss_i = ss_i + ic * ic
                d_ui = d_ui + uc * ic
                new_ss_n, new_d_un = [], []
                for n in range(_NEG):
                    nc = plsc.load_gather(nb, [nrow_idx[n], cd])
                    new_ss_n.append(ss_n[n] + nc * nc)
                    new_d_un.append(d_un[n] + uc * nc)
                return ss_u, ss_i, d_ui, tuple(new_ss_n), tuple(new_d_un)

            init = (zv, zv, zv, (zv,) * _NEG, (zv,) * _NEG)
            ss_u, ss_i, d_ui, ss_n, d_un = lax.fori_loop(0, _D, dbody, init)

            r_u = _rsqrt(jnp.maximum(ss_u, 1e-24))
            r_i = _rsqrt(jnp.maximum(ss_i, 1e-24))
            pos = d_ui * r_u * r_i
            acc_pos = acc_pos + jnp.maximum(1.0 - pos, 0.0)
            for n in range(_NEG):
                r_n = _rsqrt(jnp.maximum(ss_n[n], 1e-24))
                acc_neg = acc_neg + jnp.maximum(d_un[n] * r_u * r_n - _MARGIN, 0.0)
            return acc_pos, acc_neg

        start_group(0, 0)

        def gbody(it, carry):
            acc_pos, acc_neg = carry
            g0 = it * 2
            start_group(g0 + 1, 1)
            wait_group(g0, 0)
            acc_pos, acc_neg = compute_group(0, acc_pos, acc_neg)

            @pl.when(it + 1 < _NG // 2)
            def _():
                start_group(g0 + 2, 0)

            wait_group(g0 + 1, 1)
            acc_pos, acc_neg = compute_group(1, acc_pos, acc_neg)
            return acc_pos, acc_neg

        acc_pos, acc_neg = lax.fori_loop(0, _NG // 2, gbody, (zv, zv))

        pvec[...] = acc_pos * (1.0 / _B) + acc_neg
        pltpu.sync_copy(pvec, out_hbm.at[wid])

    return sc_kernel


_sc_kernel = _make_sc_kernel()


def kernel(u, i, neg_idx, user_emb, item_emb):
    u = u.astype(jnp.int32)
    i = i.astype(jnp.int32)
    neg_idx = neg_idx.astype(jnp.int32)
    partials = _sc_kernel(u, i, neg_idx, user_emb, item_emb)
    return jnp.sum(partials)
